# baseline (speedup 1.0000x reference)
import functools

import jax
import jax.numpy as jnp
from jax import lax
from jax.experimental import pallas as pl
from jax.experimental.pallas import tpu as pltpu
from jax.experimental.pallas import tpu_sc as plsc

N = 10000
E = 320000
D = 128

NC = 2
NS = 16
NW = NC * NS

C = 128
CPT = 80
HC = CPT // 2
E_PAD = NW * CPT * C

ACC_ROWS = 10240
ROWS_PER_TILE = ACC_ROWS // NS
DUMMY = N


def _sc_body(row_hbm, col_hbm, x_hbm, out_hbm, row_all, col_all, buf0, buf1,
             acc, gsem0, gsem1, ssem0, ssem1):
    bufs = (buf0, buf1)
    gsem = (gsem0, gsem1)
    ssem = (ssem0, ssem1)
    c = lax.axis_index("c")
    s = lax.axis_index("s")
    wid = c * NS + s

    def _zero_row(r, carry):
        for j in range(D // 16):
            buf0[r, pl.ds(j * 16, 16)] = jnp.zeros((16,), jnp.float32)
        return carry

    lax.fori_loop(0, C, _zero_row, 0, unroll=False)
    for b in range(ROWS_PER_TILE // C):
        pltpu.sync_copy(buf0, acc.at[pl.ds(s * ROWS_PER_TILE + b * C, C)])
    plsc.subcore_barrier()

    def _gather(j, b):
        pltpu.async_copy(x_hbm.at[col_all.at[j]], bufs[b], gsem[b])

    def _gather_wait(j, b):
        pltpu.make_async_copy(x_hbm.at[col_all.at[j]], bufs[b],
                              gsem[b]).wait()

    def _scatter(j, b):
        pltpu.async_copy(bufs[b], acc.at[row_all.at[j]], ssem[b], add=True)

    def _scatter_wait(j, b):
        pltpu.make_async_copy(bufs[b], acc.at[row_all.at[j]],
                              ssem[b]).wait()

    for h in range(2):
        pltpu.sync_copy(row_hbm.at[pl.ds(wid * CPT + h * HC, HC)], row_all)
        pltpu.sync_copy(col_hbm.at[pl.ds(wid * CPT + h * HC, HC)], col_all)

        def _fix_row(r, carry):
            for j in range(C // 16):
                rv = row_all[r, pl.ds(j * 16, 16)]
                cv = col_all[r, pl.ds(j * 16, 16)]
                row_all[r, pl.ds(j * 16, 16)] = jnp.where(rv == cv, DUMMY, rv)
            return carry

        lax.fori_loop(0, HC, _fix_row, 0, unroll=False)

        _gather(0, 0)
        _gather(1, 1)
        _gather_wait(0, 0)
        _scatter(0, 0)

        def _pair(j2, carry):
            for b in range(2):
                j = 2 * j2 + b
                _scatter_wait(j - 2, b)
                _gather(j, b)
                _gather_wait(j - 1, 1 - b)
                _scatter(j - 1, 1 - b)
            return carry

        lax.fori_loop(1, HC // 2, _pair, 0, unroll=False)

        _gather_wait(HC - 1, 1)
        _scatter(HC - 1, 1)
        _scatter_wait(HC - 2, 0)
        _scatter_wait(HC - 1, 1)

    plsc.subcore_barrier()

    for b in range(ROWS_PER_TILE // C):
        off = s * ROWS_PER_TILE + b * C
        pltpu.sync_copy(acc.at[pl.ds(off, C)], out_hbm.at[c, pl.ds(off, C)])


_sc_aggregate = functools.partial(
    pl.kernel,
    mesh=plsc.VectorSubcoreMesh(core_axis_name="c", subcore_axis_name="s"),
    out_type=jax.ShapeDtypeStruct((NC, ACC_ROWS, D), jnp.float32),
    scratch_types=[
        pltpu.VMEM((HC, C), jnp.int32),
        pltpu.VMEM((HC, C), jnp.int32),
        pltpu.VMEM((C, D), jnp.float32),
        pltpu.VMEM((C, D), jnp.float32),
        pltpu.VMEM_SHARED((ACC_ROWS, D), jnp.float32),
    ] + [pltpu.SemaphoreType.DMA] * 4,
)(_sc_body)


def _mlp_body(x_ref, p_ref, w1_ref, b1_ref, w2_ref, b2_ref, o_ref):
    out = x_ref[...] + p_ref[0] + p_ref[1]
    h = jnp.dot(out, w1_ref[...], preferred_element_type=jnp.float32)
    h = jnp.maximum(h + b1_ref[...], 0.0)
    y = jnp.dot(h, w2_ref[...], preferred_element_type=jnp.float32)
    o_ref[...] = y + b2_ref[...]


MB = 2000


def _mlp(x, partials, W1, b1, W2, b2):
    grid = (N // MB,)
    return pl.pallas_call(
        _mlp_body,
        grid=grid,
        in_specs=[
            pl.BlockSpec((MB, D), lambda i: (i, 0)),
            pl.BlockSpec((NC, MB, D), lambda i: (0, i, 0)),
            pl.BlockSpec((D, D), lambda i: (0, 0)),
            pl.BlockSpec((1, D), lambda i: (0, 0)),
            pl.BlockSpec((D, D), lambda i: (0, 0)),
            pl.BlockSpec((1, D), lambda i: (0, 0)),
        ],
        out_specs=pl.BlockSpec((MB, D), lambda i: (i, 0)),
        out_shape=jax.ShapeDtypeStruct((N, D), jnp.float32),
    )(x, partials, W1, b1.reshape(1, D), W2, b2.reshape(1, D))


def kernel(x, edge_index, W1, b1, W2, b2):
    row = edge_index[0].astype(jnp.int32)
    col = edge_index[1].astype(jnp.int32)
    pad = E_PAD - E
    rowp = jnp.concatenate(
        [row, jnp.full((pad,), DUMMY, jnp.int32)]).reshape(E_PAD // C, C)
    colp = jnp.concatenate(
        [col, jnp.zeros((pad,), jnp.int32)]).reshape(E_PAD // C, C)
    partials = _sc_aggregate(rowp, colp, x)
    return _mlp(x, partials, W1, b1, W2, b2)
